# baseline (speedup 1.0000x reference)
import functools

import jax
import jax.numpy as jnp
from jax import lax
from jax.experimental import pallas as pl
from jax.experimental.pallas import tpu as pltpu
from jax.experimental.pallas import tpu_sc as plsc


def kernel(x, table):
    bs, seq_len = x.shape
    num_rows, emb = table.shape

    info = plsc.get_sparse_core_info()
    nw = info.num_cores * info.num_subcores
    rows_per = seq_len // nw

    mesh = plsc.VectorSubcoreMesh(core_axis_name="c", subcore_axis_name="s")

    chunk = 16
    nbuf = 4
    nchunks = rows_per // chunk

    @functools.partial(
        pl.kernel,
        mesh=mesh,
        out_type=jax.ShapeDtypeStruct((seq_len, emb), table.dtype),
        scratch_types=[
            pltpu.VMEM((nbuf, chunk, emb), table.dtype),
        ]
        + [pltpu.SemaphoreType.DMA] * (2 * nbuf),
    )
    def positional_lookup(table_hbm, out_hbm, buf, *sems):
        wid = lax.axis_index("s") * info.num_cores + lax.axis_index("c")
        base = wid * rows_per
        sin = sems[:nbuf]
        sout = sems[nbuf:]

        def in_copy(g, b):
            return pltpu.make_async_copy(
                table_hbm.at[pl.ds(base + g * chunk, chunk)], buf.at[b], sin[b]
            )

        def out_copy(g, b):
            return pltpu.make_async_copy(
                buf.at[b], out_hbm.at[pl.ds(base + g * chunk, chunk)], sout[b]
            )

        in_copy(0, 0).start()
        for g in range(nchunks):
            b = g % nbuf
            in_copy(g, b).wait()
            c = g + 1
            if c < nchunks:
                if c >= nbuf:
                    out_copy(c - nbuf, c % nbuf).wait()
                in_copy(c, c % nbuf).start()
            out_copy(g, b).start()
        for g in range(max(0, nchunks - nbuf), nchunks):
            out_copy(g, g % nbuf).wait()

    return positional_lookup(table)


# speedup vs baseline: 1.1283x; 1.1283x over previous
import functools

import jax
import jax.numpy as jnp
from jax import lax
from jax.experimental import pallas as pl
from jax.experimental.pallas import tpu as pltpu
from jax.experimental.pallas import tpu_sc as plsc


def kernel(x, table):
    bs, seq_len = x.shape
    num_rows, emb = table.shape

    info = plsc.get_sparse_core_info()
    nw = info.num_cores * info.num_subcores
    rows_per = seq_len // nw

    mesh = plsc.VectorSubcoreMesh(core_axis_name="c", subcore_axis_name="s")

    chunk = 32
    nbuf = 3
    nchunks = rows_per // chunk

    @functools.partial(
        pl.kernel,
        mesh=mesh,
        out_type=jax.ShapeDtypeStruct((seq_len, emb), table.dtype),
        scratch_types=[
            pltpu.VMEM((nbuf, chunk, emb), table.dtype),
        ]
        + [pltpu.SemaphoreType.DMA] * (2 * nbuf),
    )
    def positional_lookup(table_hbm, out_hbm, buf, *sems):
        wid = lax.axis_index("s") * info.num_cores + lax.axis_index("c")
        base = wid * rows_per
        sin = sems[:nbuf]
        sout = sems[nbuf:]

        def in_copy(g, b):
            return pltpu.make_async_copy(
                table_hbm.at[pl.ds(base + g * chunk, chunk)], buf.at[b], sin[b]
            )

        def out_copy(g, b):
            return pltpu.make_async_copy(
                buf.at[b], out_hbm.at[pl.ds(base + g * chunk, chunk)], sout[b]
            )

        in_copy(0, 0).start()
        for g in range(nchunks):
            b = g % nbuf
            in_copy(g, b).wait()
            c = g + 1
            if c < nchunks:
                if c >= nbuf:
                    out_copy(c - nbuf, c % nbuf).wait()
                in_copy(c, c % nbuf).start()
            out_copy(g, b).start()
        for g in range(max(0, nchunks - nbuf), nchunks):
            out_copy(g, g % nbuf).wait()

    return positional_lookup(table)
